# baseline (speedup 1.0000x reference)
import functools

import jax
import jax.numpy as jnp
from jax import lax
from jax.experimental import pallas as pl
from jax.experimental.pallas import tpu as pltpu
from jax.experimental.pallas import tpu_sc as plsc

B = 16384
K = 20
D = 16
NC = 2
NS = 16
NW = NC * NS
BPW = B // NW
NEG_PER_W = BPW * K
NCH = 2048

_mesh = plsc.VectorSubcoreMesh(core_axis_name="c", subcore_axis_name="s")


@functools.partial(
    pl.kernel,
    mesh=_mesh,
    out_type=(
        jax.ShapeDtypeStruct((B, D), jnp.float32),
        jax.ShapeDtypeStruct((B, D), jnp.float32),
        jax.ShapeDtypeStruct((B * K, D), jnp.float32),
    ),
    scratch_types=[
        pltpu.VMEM((BPW,), jnp.int32),
        pltpu.VMEM((BPW, D), jnp.float32),
        pltpu.VMEM((NCH,), jnp.int32),
        pltpu.VMEM((NCH, D), jnp.float32),
        pltpu.SemaphoreType.DMA,
    ],
    compiler_params=pltpu.CompilerParams(use_tc_tiling_on_sc=False),
)
def _sc_gather(cw_hbm, pc_hbm, neg_hbm, wtab_hbm, ctab_hbm,
               w_out, c_out, n_out,
               idx_v, rows_v, nidx_v, nrows_v, sem):
    wid = lax.axis_index("s") * NC + lax.axis_index("c")
    base = wid * BPW

    pltpu.sync_copy(cw_hbm.at[pl.ds(base, BPW)], idx_v)
    pltpu.async_copy(wtab_hbm.at[idx_v], rows_v, sem).wait()
    pltpu.sync_copy(rows_v, w_out.at[pl.ds(base, BPW)])

    pltpu.sync_copy(pc_hbm.at[pl.ds(base, BPW)], idx_v)
    pltpu.async_copy(ctab_hbm.at[idx_v], rows_v, sem).wait()
    pltpu.sync_copy(rows_v, c_out.at[pl.ds(base, BPW)])

    nbase = wid * NEG_PER_W
    for ch in range(NEG_PER_W // NCH):
        off = nbase + ch * NCH
        pltpu.sync_copy(neg_hbm.at[pl.ds(off, NCH)], nidx_v)
        pltpu.async_copy(ctab_hbm.at[nidx_v], nrows_v, sem).wait()
        pltpu.sync_copy(nrows_v, n_out.at[pl.ds(off, NCH)])


BB = 2048
NBLK = B // BB


def _log_sigmoid(x):
    return jnp.minimum(x, 0.0) - jnp.log1p(jnp.exp(-jnp.abs(x)))


def _tc_body(w_ref, c_ref, n_ref, out_ref):
    w = w_ref[...]
    c = c_ref[...]
    n = n_ref[...].reshape(BB, K, D)
    pos_score = jnp.sum(w * c, axis=1)
    neg_score = jnp.sum(n * w[:, None, :], axis=2)
    blk = jnp.sum(_log_sigmoid(pos_score)) + jnp.sum(_log_sigmoid(-neg_score))

    @pl.when(pl.program_id(0) == 0)
    def _():
        out_ref[...] = jnp.zeros_like(out_ref)

    out_ref[...] += (blk * (-1.0 / B)).reshape(1, 1)


def _tc_loss(w_g, c_g, n_g):
    return pl.pallas_call(
        _tc_body,
        grid=(NBLK,),
        in_specs=[
            pl.BlockSpec((BB, D), lambda i: (i, 0)),
            pl.BlockSpec((BB, D), lambda i: (i, 0)),
            pl.BlockSpec((BB * K, D), lambda i: (i, 0)),
        ],
        out_specs=pl.BlockSpec((1, 1), lambda i: (0, 0)),
        out_shape=jax.ShapeDtypeStruct((1, 1), jnp.float32),
    )(w_g, c_g, n_g)


def kernel(centrals_words, pos_context, neg_context, word_emb, con_emb):
    neg_flat = neg_context.reshape(B * K)
    w_g, c_g, n_g = _sc_gather(centrals_words, pos_context, neg_flat,
                               word_emb, con_emb)
    loss = _tc_loss(w_g, c_g, n_g)
    return loss[0, 0]


# speedup vs baseline: 1.1825x; 1.1825x over previous
import functools

import jax
import jax.numpy as jnp
from jax import lax
from jax.experimental import pallas as pl
from jax.experimental.pallas import tpu as pltpu
from jax.experimental.pallas import tpu_sc as plsc

B = 16384
K = 20
D = 16
RPL = 128 // D
TROWS = 1000000 // RPL
NC = 2
NS = 16
NW = NC * NS
BPW = B // NW
CB = 16
NCHUNK = BPW // CB

_mesh = plsc.VectorSubcoreMesh(core_axis_name="c", subcore_axis_name="s")


@functools.partial(
    pl.kernel,
    mesh=_mesh,
    out_type=(
        jax.ShapeDtypeStruct((B,), jnp.float32),
        jax.ShapeDtypeStruct((B * K,), jnp.float32),
    ),
    scratch_types=[
        pltpu.VMEM((BPW,), jnp.int32),
        pltpu.VMEM((BPW,), jnp.int32),
        pltpu.VMEM((BPW * K,), jnp.int32),
        pltpu.VMEM((CB,), jnp.int32),
        pltpu.VMEM((CB,), jnp.int32),
        pltpu.VMEM((CB * K,), jnp.int32),
        pltpu.VMEM((CB, 128), jnp.float32),
        pltpu.VMEM((CB, 128), jnp.float32),
        pltpu.VMEM((CB * K, 128), jnp.float32),
        pltpu.VMEM((BPW,), jnp.float32),
        pltpu.VMEM((BPW * K,), jnp.float32),
        pltpu.SemaphoreType.DMA,
    ],
    compiler_params=pltpu.CompilerParams(needs_layout_passes=False),
)
def _sc_scores(cw_hbm, pc_hbm, neg_hbm, wtab_hbm, ctab_hbm,
               pos_out, neg_out,
               cwi_v, pci_v, ngi_v, cwg_v, pcg_v, ngg_v,
               wbuf, cbuf, nbuf, psc_v, nsc_v, sem):
    wid = lax.axis_index("s") * NC + lax.axis_index("c")
    base = wid * BPW

    pltpu.sync_copy(cw_hbm.at[pl.ds(base, BPW)], cwi_v)
    pltpu.sync_copy(pc_hbm.at[pl.ds(base, BPW)], pci_v)
    pltpu.sync_copy(neg_hbm.at[pl.ds(base * K, BPW * K)], ngi_v)

    lanes = lax.iota(jnp.int32, CB)

    def chunk(ch, carry):
        cb = ch * CB
        cwi = cwi_v[pl.ds(cb, CB)]
        pci = pci_v[pl.ds(cb, CB)]
        cwg_v[...] = lax.shift_right_logical(cwi, RPL.bit_length() - 1)
        pcg_v[...] = lax.shift_right_logical(pci, RPL.bit_length() - 1)
        for t in range(CB * K // 16):
            ngg_v[pl.ds(t * 16, 16)] = lax.shift_right_logical(
                ngi_v[pl.ds(cb * K + t * 16, 16)], RPL.bit_length() - 1)
        cpw = pltpu.async_copy(wtab_hbm.at[cwg_v], wbuf, sem)
        cpc = pltpu.async_copy(ctab_hbm.at[pcg_v], cbuf, sem)
        cpn = pltpu.async_copy(ctab_hbm.at[ngg_v], nbuf, sem)
        cpw.wait()
        cpc.wait()
        cpn.wait()

        wsub = (cwi & (RPL - 1)) * D
        csub = (pci & (RPL - 1)) * D
        acc = jnp.zeros((CB,), jnp.float32)
        wj = []
        for j in range(D):
            wv = plsc.load_gather(wbuf, [lanes, wsub + j])
            cv = plsc.load_gather(cbuf, [lanes, csub + j])
            wj.append(wv)
            acc = acc + wv * cv
        psc_v[pl.ds(cb, CB)] = acc

        for k in range(K):
            rows_nk = lanes * K + k
            nsub = (plsc.load_gather(ngi_v, [cb * K + rows_nk]) & (RPL - 1)) * D
            accn = jnp.zeros((CB,), jnp.float32)
            for j in range(D):
                nv = plsc.load_gather(nbuf, [rows_nk, nsub + j])
                accn = accn + nv * wj[j]
            plsc.store_scatter(nsc_v, [cb * K + rows_nk], accn)
        return carry

    lax.fori_loop(0, NCHUNK, chunk, 0)

    pltpu.sync_copy(psc_v, pos_out.at[pl.ds(base, BPW)])
    pltpu.sync_copy(nsc_v, neg_out.at[pl.ds(base * K, BPW * K)])


def _log_sigmoid(x):
    return jnp.minimum(x, 0.0) - jnp.log1p(jnp.exp(-jnp.abs(x)))


def _tc_body(p_ref, n_ref, out_ref):
    s = jnp.sum(_log_sigmoid(p_ref[...])) + jnp.sum(_log_sigmoid(-n_ref[...]))
    out_ref[...] = (s * (-1.0 / B)).reshape(1, 1)


def _tc_loss(pos_score, neg_score):
    return pl.pallas_call(
        _tc_body,
        out_shape=jax.ShapeDtypeStruct((1, 1), jnp.float32),
    )(pos_score.reshape(B // 128, 128), neg_score.reshape(B * K // 128, 128))


def kernel(centrals_words, pos_context, neg_context, word_emb, con_emb):
    neg_flat = neg_context.reshape(B * K)
    wt = word_emb.reshape(TROWS, 128)
    ct = con_emb.reshape(TROWS, 128)
    pos_score, neg_score = _sc_scores(centrals_words, pos_context, neg_flat,
                                      wt, ct)
    loss = _tc_loss(pos_score, neg_score)
    return loss[0, 0]


# speedup vs baseline: 3.5953x; 3.0404x over previous
import functools

import jax
import jax.numpy as jnp
from jax import lax
from jax.experimental import pallas as pl
from jax.experimental.pallas import tpu as pltpu
from jax.experimental.pallas import tpu_sc as plsc

B = 16384
K = 20
D = 16
RPL = 128 // D
SLINES = 131072
LOGS = 17
NC = 2
NS = 16
NW = NC * NS
BPW = B // NW
CB = 16
NCHUNK = BPW // CB

_mesh = plsc.VectorSubcoreMesh(core_axis_name="c", subcore_axis_name="s")


@functools.partial(
    pl.kernel,
    mesh=_mesh,
    out_type=(
        jax.ShapeDtypeStruct((B,), jnp.float32),
        jax.ShapeDtypeStruct((B * K,), jnp.float32),
    ),
    scratch_types=[
        pltpu.VMEM((BPW,), jnp.int32),
        pltpu.VMEM((BPW,), jnp.int32),
        pltpu.VMEM((BPW * K,), jnp.int32),
        pltpu.VMEM((CB,), jnp.int32),
        pltpu.VMEM((CB,), jnp.int32),
        pltpu.VMEM((CB * K,), jnp.int32),
        pltpu.VMEM((CB, 128), jnp.float32),
        pltpu.VMEM((CB, 128), jnp.float32),
        pltpu.VMEM((CB * K, 128), jnp.float32),
        pltpu.VMEM((BPW,), jnp.float32),
        pltpu.VMEM((BPW * K,), jnp.float32),
        pltpu.SemaphoreType.DMA,
    ],
    compiler_params=pltpu.CompilerParams(needs_layout_passes=False),
)
def _sc_scores(cw_hbm, pc_hbm, neg_hbm, wtab_hbm, ctab_hbm,
               pos_out, neg_out,
               cwi_v, pci_v, ngi_v, cwg_v, pcg_v, ngg_v,
               wbuf, cbuf, nbuf, psc_v, nsc_v, sem):
    wid = lax.axis_index("s") * NC + lax.axis_index("c")
    base = wid * BPW

    pltpu.sync_copy(cw_hbm.at[pl.ds(base, BPW)], cwi_v)
    pltpu.sync_copy(pc_hbm.at[pl.ds(base, BPW)], pci_v)
    pltpu.sync_copy(neg_hbm.at[pl.ds(base * K, BPW * K)], ngi_v)

    lanes = lax.iota(jnp.int32, CB)

    def chunk(ch, carry):
        cb = ch * CB
        cwi = cwi_v[pl.ds(cb, CB)]
        pci = pci_v[pl.ds(cb, CB)]
        cwg_v[...] = cwi & (SLINES - 1)
        pcg_v[...] = pci & (SLINES - 1)
        for t in range(CB * K // 16):
            ngg_v[pl.ds(t * 16, 16)] = ngi_v[pl.ds(cb * K + t * 16, 16)] & (SLINES - 1)
        cpw = pltpu.async_copy(wtab_hbm.at[cwg_v], wbuf, sem)
        cpc = pltpu.async_copy(ctab_hbm.at[pcg_v], cbuf, sem)
        cpn = pltpu.async_copy(ctab_hbm.at[ngg_v], nbuf, sem)
        cpw.wait()
        cpc.wait()
        cpn.wait()

        wsub = lax.shift_right_logical(cwi, LOGS) * D
        csub = lax.shift_right_logical(pci, LOGS) * D
        acc = jnp.zeros((CB,), jnp.float32)
        wj = []
        for j in range(D):
            wv = plsc.load_gather(wbuf, [lanes, wsub + j])
            cv = plsc.load_gather(cbuf, [lanes, csub + j])
            wj.append(wv)
            acc = acc + wv * cv
        psc_v[pl.ds(cb, CB)] = acc

        for k in range(K):
            rows_nk = lanes * K + k
            nsub = lax.shift_right_logical(
                plsc.load_gather(ngi_v, [cb * K + rows_nk]), LOGS) * D
            accn = jnp.zeros((CB,), jnp.float32)
            for j in range(D):
                nv = plsc.load_gather(nbuf, [rows_nk, nsub + j])
                accn = accn + nv * wj[j]
            plsc.store_scatter(nsc_v, [cb * K + rows_nk], accn)
        return carry

    lax.fori_loop(0, NCHUNK, chunk, 0)

    pltpu.sync_copy(psc_v, pos_out.at[pl.ds(base, BPW)])
    pltpu.sync_copy(nsc_v, neg_out.at[pl.ds(base * K, BPW * K)])


TW = 1024
TGRID = 128


def _tr_body(*refs):
    wrefs = refs[:RPL]
    crefs = refs[RPL:2 * RPL]
    wo_ref, co_ref = refs[2 * RPL:]
    ident = jnp.eye(128, dtype=jnp.float32)
    for srcs, dst in ((wrefs, wo_ref), (crefs, co_ref)):
        xs = jnp.concatenate([r[...] for r in srcs], axis=0)
        dst[...] = lax.dot_general(xs, ident, (((0,), (0,)), ((), ())))


def _transpose_tables(wt_t, ct_t):
    last = 1000000 // TW
    in_spec = [
        pl.BlockSpec((D, TW), lambda g, a=a: (0, jnp.minimum(a * TGRID + g, last)))
        for a in range(RPL)
    ]
    return pl.pallas_call(
        _tr_body,
        grid=(TGRID,),
        in_specs=in_spec + in_spec,
        out_specs=[
            pl.BlockSpec((TW, 128), lambda g: (g, 0)),
            pl.BlockSpec((TW, 128), lambda g: (g, 0)),
        ],
        out_shape=[
            jax.ShapeDtypeStruct((SLINES, 128), jnp.float32),
            jax.ShapeDtypeStruct((SLINES, 128), jnp.float32),
        ],
    )(*([wt_t] * RPL + [ct_t] * RPL))


def _log_sigmoid(x):
    return jnp.minimum(x, 0.0) - jnp.log1p(jnp.exp(-jnp.abs(x)))


def _tc_body(p_ref, n_ref, out_ref):
    s = jnp.sum(_log_sigmoid(p_ref[...])) + jnp.sum(_log_sigmoid(-n_ref[...]))
    out_ref[...] = (s * (-1.0 / B)).reshape(1, 1)


def _tc_loss(pos_score, neg_score):
    return pl.pallas_call(
        _tc_body,
        out_shape=jax.ShapeDtypeStruct((1, 1), jnp.float32),
    )(pos_score.reshape(B // 128, 128), neg_score.reshape(B * K // 128, 128))


def kernel(centrals_words, pos_context, neg_context, word_emb, con_emb):
    neg_flat = neg_context.reshape(B * K)
    wt, ct = _transpose_tables(word_emb.T, con_emb.T)
    pos_score, neg_score = _sc_scores(centrals_words, pos_context, neg_flat,
                                      wt, ct)
    loss = _tc_loss(pos_score, neg_score)
    return loss[0, 0]


# speedup vs baseline: 4.3455x; 1.2087x over previous
import functools

import jax
import jax.numpy as jnp
from jax import lax
from jax.experimental import pallas as pl
from jax.experimental.pallas import tpu as pltpu
from jax.experimental.pallas import tpu_sc as plsc

B = 16384
K = 20
D = 16
RPL = 128 // D
SLINES = 131072
LOGS = 17
NC = 2
NS = 16
NW = NC * NS
BPW = B // NW
CB = 16
NCHUNK = BPW // CB

_mesh = plsc.VectorSubcoreMesh(core_axis_name="c", subcore_axis_name="s")


@functools.partial(
    pl.kernel,
    mesh=_mesh,
    out_type=(
        jax.ShapeDtypeStruct((B,), jnp.float32),
        jax.ShapeDtypeStruct((B * K,), jnp.float32),
    ),
    scratch_types=[
        pltpu.VMEM((BPW,), jnp.int32),
        pltpu.VMEM((BPW,), jnp.int32),
        pltpu.VMEM((BPW * K,), jnp.int32),
        pltpu.VMEM((CB,), jnp.int32),
        pltpu.VMEM((CB,), jnp.int32),
        pltpu.VMEM((CB,), jnp.int32),
        pltpu.VMEM((CB,), jnp.int32),
        pltpu.VMEM((CB * K,), jnp.int32),
        pltpu.VMEM((CB * K,), jnp.int32),
        pltpu.VMEM((CB, 128), jnp.float32),
        pltpu.VMEM((CB, 128), jnp.float32),
        pltpu.VMEM((CB, 128), jnp.float32),
        pltpu.VMEM((CB, 128), jnp.float32),
        pltpu.VMEM((CB * K, 128), jnp.float32),
        pltpu.VMEM((CB * K, 128), jnp.float32),
        pltpu.VMEM((BPW,), jnp.float32),
        pltpu.VMEM((BPW * K,), jnp.float32),
        pltpu.SemaphoreType.DMA,
        pltpu.SemaphoreType.DMA,
    ],
    compiler_params=pltpu.CompilerParams(needs_layout_passes=False),
)
def _sc_scores(cw_hbm, pc_hbm, neg_hbm, wtab_hbm, ctab_hbm,
               pos_out, neg_out,
               cwi_v, pci_v, ngi_v, cwg0, cwg1, pcg0, pcg1, ngg0, ngg1,
               wbuf0, wbuf1, cbuf0, cbuf1, nbuf0, nbuf1,
               psc_v, nsc_v, sem0, sem1):
    wid = lax.axis_index("s") * NC + lax.axis_index("c")
    base = wid * BPW

    pltpu.sync_copy(cw_hbm.at[pl.ds(base, BPW)], cwi_v)
    pltpu.sync_copy(pc_hbm.at[pl.ds(base, BPW)], pci_v)
    for k in range(K):
        pltpu.sync_copy(neg_hbm.at[pl.ds(k * B + base, BPW)],
                        ngi_v.at[pl.ds(k * BPW, BPW)])

    lanes = lax.iota(jnp.int32, CB)
    bufs = (
        (cwg0, pcg0, ngg0, wbuf0, cbuf0, nbuf0, sem0),
        (cwg1, pcg1, ngg1, wbuf1, cbuf1, nbuf1, sem1),
    )

    def fire(ch, slot):
        cwg, pcg, ngg, wb, cb_, nb, sem = bufs[slot]
        cb = ch * CB
        cwg[...] = cwi_v[pl.ds(cb, CB)] & (SLINES - 1)
        pcg[...] = pci_v[pl.ds(cb, CB)] & (SLINES - 1)
        for k in range(K):
            ngg[pl.ds(k * CB, CB)] = \
                ngi_v[pl.ds(k * BPW + cb, CB)] & (SLINES - 1)
        return (pltpu.async_copy(wtab_hbm.at[cwg], wb, sem),
                pltpu.async_copy(ctab_hbm.at[pcg], cb_, sem),
                pltpu.async_copy(ctab_hbm.at[ngg], nb, sem))

    def compute(ch, slot, copies):
        _, _, _, wb, cb_, nb, sem = bufs[slot]
        for c in copies:
            c.wait()
        cb = ch * CB
        cwi = cwi_v[pl.ds(cb, CB)]
        pci = pci_v[pl.ds(cb, CB)]
        wsub = lax.shift_right_logical(cwi, LOGS) * D
        csub = lax.shift_right_logical(pci, LOGS) * D
        acc = jnp.zeros((CB,), jnp.float32)
        wj = []
        for j in range(D):
            wv = plsc.load_gather(wb, [lanes, wsub + j])
            cv = plsc.load_gather(cb_, [lanes, csub + j])
            wj.append(wv)
            acc = acc + wv * cv
        psc_v[pl.ds(cb, CB)] = acc

        for k in range(K):
            rows_nk = k * CB + lanes
            nsub = lax.shift_right_logical(
                ngi_v[pl.ds(k * BPW + cb, CB)], LOGS) * D
            accn = jnp.zeros((CB,), jnp.float32)
            for j in range(D):
                nv = plsc.load_gather(nb, [rows_nk, nsub + j])
                accn = accn + nv * wj[j]
            nsc_v[pl.ds(cb * K + k * CB, CB)] = accn

    def pair(g, carry):
        ch = g * 2
        c_a = fire(ch, 0)
        c_b = fire(ch + 1, 1)
        compute(ch, 0, c_a)
        compute(ch + 1, 1, c_b)
        return carry

    lax.fori_loop(0, NCHUNK // 2, pair, 0)

    pltpu.sync_copy(psc_v, pos_out.at[pl.ds(base, BPW)])
    pltpu.sync_copy(nsc_v, neg_out.at[pl.ds(base * K, BPW * K)])


TW = 4096
TGRID = SLINES // TW


def _tr_body(*refs):
    wrefs = refs[:RPL]
    crefs = refs[RPL:2 * RPL]
    wo_ref, co_ref = refs[2 * RPL:]
    ident = jnp.eye(128, dtype=jnp.float32)
    for srcs, dst in ((wrefs, wo_ref), (crefs, co_ref)):
        xs = jnp.concatenate([r[...] for r in srcs], axis=0)
        dst[...] = lax.dot_general(xs, ident, (((0,), (0,)), ((), ())))


def _transpose_tables(wt_t, ct_t):
    last = 1000000 // TW
    in_spec = [
        pl.BlockSpec((D, TW), lambda g, a=a: (0, jnp.minimum(a * TGRID + g, last)))
        for a in range(RPL)
    ]
    return pl.pallas_call(
        _tr_body,
        grid=(TGRID,),
        in_specs=in_spec + in_spec,
        out_specs=[
            pl.BlockSpec((TW, 128), lambda g: (g, 0)),
            pl.BlockSpec((TW, 128), lambda g: (g, 0)),
        ],
        out_shape=[
            jax.ShapeDtypeStruct((SLINES, 128), jnp.float32),
            jax.ShapeDtypeStruct((SLINES, 128), jnp.float32),
        ],
    )(*([wt_t] * RPL + [ct_t] * RPL))


def _log_sigmoid(x):
    return jnp.minimum(x, 0.0) - jnp.log1p(jnp.exp(-jnp.abs(x)))


def _tc_body(p_ref, n_ref, out_ref):
    s = jnp.sum(_log_sigmoid(p_ref[...])) + jnp.sum(_log_sigmoid(-n_ref[...]))
    out_ref[...] = (s * (-1.0 / B)).reshape(1, 1)


def _tc_loss(pos_score, neg_score):
    return pl.pallas_call(
        _tc_body,
        out_shape=jax.ShapeDtypeStruct((1, 1), jnp.float32),
    )(pos_score.reshape(B // 128, 128), neg_score.reshape(B * K // 128, 128))


def kernel(centrals_words, pos_context, neg_context, word_emb, con_emb):
    neg_km = neg_context.T.reshape(B * K)
    wt, ct = _transpose_tables(word_emb.T, con_emb.T)
    pos_score, neg_score = _sc_scores(centrals_words, pos_context, neg_km,
                                      wt, ct)
    loss = _tc_loss(pos_score, neg_score)
    return loss[0, 0]


# speedup vs baseline: 4.5002x; 1.0356x over previous
import functools

import jax
import jax.numpy as jnp
from jax import lax
from jax.experimental import pallas as pl
from jax.experimental.pallas import tpu as pltpu
from jax.experimental.pallas import tpu_sc as plsc

B = 16384
K = 20
D = 16
RPL = 128 // D
SLINES = 131072
LOGS = 17
NC = 2
NS = 16
NW = NC * NS
BPW = B // NW
CB = 16
NCHUNK = BPW // CB

_mesh = plsc.VectorSubcoreMesh(core_axis_name="c", subcore_axis_name="s")


@functools.partial(
    pl.kernel,
    mesh=_mesh,
    out_type=(
        jax.ShapeDtypeStruct((B,), jnp.float32),
        jax.ShapeDtypeStruct((B * K,), jnp.float32),
    ),
    scratch_types=[
        pltpu.VMEM((BPW,), jnp.int32),
        pltpu.VMEM((BPW,), jnp.int32),
        pltpu.VMEM((BPW * K,), jnp.int32),
        pltpu.VMEM((CB,), jnp.int32),
        pltpu.VMEM((CB,), jnp.int32),
        pltpu.VMEM((CB,), jnp.int32),
        pltpu.VMEM((CB,), jnp.int32),
        pltpu.VMEM((CB * K,), jnp.int32),
        pltpu.VMEM((CB * K,), jnp.int32),
        pltpu.VMEM((CB, 128), jnp.float32),
        pltpu.VMEM((CB, 128), jnp.float32),
        pltpu.VMEM((CB, 128), jnp.float32),
        pltpu.VMEM((CB, 128), jnp.float32),
        pltpu.VMEM((CB * K, 128), jnp.float32),
        pltpu.VMEM((CB * K, 128), jnp.float32),
        pltpu.VMEM((BPW,), jnp.float32),
        pltpu.VMEM((BPW * K,), jnp.float32),
        pltpu.SemaphoreType.DMA,
        pltpu.SemaphoreType.DMA,
    ],
    compiler_params=pltpu.CompilerParams(needs_layout_passes=False),
)
def _sc_scores(cw_hbm, pc_hbm, neg_hbm, wtab_hbm, ctab_hbm,
               pos_out, neg_out,
               cwi_v, pci_v, ngi_v, cwg0, cwg1, pcg0, pcg1, ngg0, ngg1,
               wbuf0, wbuf1, cbuf0, cbuf1, nbuf0, nbuf1,
               psc_v, nsc_v, sem0, sem1):
    wid = lax.axis_index("s") * NC + lax.axis_index("c")
    base = wid * BPW

    stage = [
        pltpu.async_copy(cw_hbm.at[pl.ds(base, BPW)], cwi_v, sem0),
        pltpu.async_copy(pc_hbm.at[pl.ds(base, BPW)], pci_v, sem0),
    ]
    stage += [
        pltpu.async_copy(neg_hbm.at[pl.ds(k * B + base, BPW)],
                         ngi_v.at[pl.ds(k * BPW, BPW)], sem0)
        for k in range(K)
    ]
    for c in stage:
        c.wait()

    lanes = lax.iota(jnp.int32, CB)
    bufs = (
        (cwg0, pcg0, ngg0, wbuf0, cbuf0, nbuf0, sem0),
        (cwg1, pcg1, ngg1, wbuf1, cbuf1, nbuf1, sem1),
    )

    def fire(ch, slot):
        cwg, pcg, ngg, wb, cb_, nb, sem = bufs[slot]
        cb = ch * CB
        cwg[...] = cwi_v[pl.ds(cb, CB)] & (SLINES - 1)
        pcg[...] = pci_v[pl.ds(cb, CB)] & (SLINES - 1)
        for k in range(K):
            ngg[pl.ds(k * CB, CB)] = \
                ngi_v[pl.ds(k * BPW + cb, CB)] & (SLINES - 1)
        return (pltpu.async_copy(wtab_hbm.at[cwg], wb, sem),
                pltpu.async_copy(ctab_hbm.at[pcg], cb_, sem),
                pltpu.async_copy(ctab_hbm.at[ngg], nb, sem))

    def compute(ch, slot, copies):
        _, _, _, wb, cb_, nb, sem = bufs[slot]
        for c in copies:
            c.wait()
        cb = ch * CB
        cwi = cwi_v[pl.ds(cb, CB)]
        pci = pci_v[pl.ds(cb, CB)]
        wsub = lax.shift_right_logical(cwi, LOGS) * D
        csub = lax.shift_right_logical(pci, LOGS) * D
        acc = jnp.zeros((CB,), jnp.float32)
        wj = []
        for j in range(D):
            wv = plsc.load_gather(wb, [lanes, wsub + j])
            cv = plsc.load_gather(cb_, [lanes, csub + j])
            wj.append(wv)
            acc = acc + wv * cv
        psc_v[pl.ds(cb, CB)] = acc

        for k in range(K):
            rows_nk = k * CB + lanes
            nsub = lax.shift_right_logical(
                ngi_v[pl.ds(k * BPW + cb, CB)], LOGS) * D
            accn = jnp.zeros((CB,), jnp.float32)
            for j in range(D):
                nv = plsc.load_gather(nb, [rows_nk, nsub + j])
                accn = accn + nv * wj[j]
            nsc_v[pl.ds(cb * K + k * CB, CB)] = accn

    def pair(g, carry):
        ch = g * 2
        c_a = fire(ch, 0)
        c_b = fire(ch + 1, 1)
        compute(ch, 0, c_a)
        compute(ch + 1, 1, c_b)
        return carry

    lax.fori_loop(0, NCHUNK // 2, pair, 0)

    pltpu.sync_copy(psc_v, pos_out.at[pl.ds(base, BPW)])
    pltpu.sync_copy(nsc_v, neg_out.at[pl.ds(base * K, BPW * K)])


TW = 4096
TGRID = SLINES // TW


def _tr_body(*refs):
    wrefs = refs[:RPL]
    crefs = refs[RPL:2 * RPL]
    wo_ref, co_ref = refs[2 * RPL:]
    ident = jnp.eye(128, dtype=jnp.float32)
    for srcs, dst in ((wrefs, wo_ref), (crefs, co_ref)):
        xs = jnp.concatenate([r[...] for r in srcs], axis=0)
        dst[...] = lax.dot_general(xs, ident, (((0,), (0,)), ((), ())))


def _transpose_tables(wt_t, ct_t):
    last = 1000000 // TW
    in_spec = [
        pl.BlockSpec((D, TW), lambda g, a=a: (0, jnp.minimum(a * TGRID + g, last)))
        for a in range(RPL)
    ]
    return pl.pallas_call(
        _tr_body,
        grid=(TGRID,),
        in_specs=in_spec + in_spec,
        out_specs=[
            pl.BlockSpec((TW, 128), lambda g: (g, 0)),
            pl.BlockSpec((TW, 128), lambda g: (g, 0)),
        ],
        out_shape=[
            jax.ShapeDtypeStruct((SLINES, 128), jnp.float32),
            jax.ShapeDtypeStruct((SLINES, 128), jnp.float32),
        ],
    )(*([wt_t] * RPL + [ct_t] * RPL))


def _log_sigmoid(x):
    return jnp.minimum(x, 0.0) - jnp.log1p(jnp.exp(-jnp.abs(x)))


def _tc_body(p_ref, n_ref, out_ref):
    s = jnp.sum(_log_sigmoid(p_ref[...])) + jnp.sum(_log_sigmoid(-n_ref[...]))
    out_ref[...] = (s * (-1.0 / B)).reshape(1, 1)


def _tc_loss(pos_score, neg_score):
    return pl.pallas_call(
        _tc_body,
        out_shape=jax.ShapeDtypeStruct((1, 1), jnp.float32),
    )(pos_score.reshape(B // 128, 128), neg_score.reshape(B * K // 128, 128))


def kernel(centrals_words, pos_context, neg_context, word_emb, con_emb):
    neg_km = neg_context.T.reshape(B * K)
    wt, ct = _transpose_tables(word_emb.T, con_emb.T)
    pos_score, neg_score = _sc_scores(centrals_words, pos_context, neg_km,
                                      wt, ct)
    loss = _tc_loss(pos_score, neg_score)
    return loss[0, 0]


# speedup vs baseline: 5.9999x; 1.3333x over previous
import functools

import jax
import jax.numpy as jnp
from jax import lax
from jax.experimental import pallas as pl
from jax.experimental.pallas import tpu as pltpu
from jax.experimental.pallas import tpu_sc as plsc

B = 16384
K = 20
D = 16
RPL = 128 // D
SLINES = 131072
LOGS = 17
NC = 2
NS = 16
NW = NC * NS
BPW = B // NW
CB = 16
NCHUNK = BPW // CB

_mesh = plsc.VectorSubcoreMesh(core_axis_name="c", subcore_axis_name="s")


@functools.partial(
    pl.kernel,
    mesh=_mesh,
    out_type=(
        jax.ShapeDtypeStruct((B,), jnp.float32),
        jax.ShapeDtypeStruct((B * K,), jnp.float32),
    ),
    scratch_types=[
        pltpu.VMEM((BPW,), jnp.int32),
        pltpu.VMEM((BPW,), jnp.int32),
        pltpu.VMEM((BPW * K,), jnp.int32),
        pltpu.VMEM((CB,), jnp.int32),
        pltpu.VMEM((CB,), jnp.int32),
        pltpu.VMEM((CB,), jnp.int32),
        pltpu.VMEM((CB,), jnp.int32),
        pltpu.VMEM((CB * K,), jnp.int32),
        pltpu.VMEM((CB * K,), jnp.int32),
        pltpu.VMEM((CB, D), jnp.float32),
        pltpu.VMEM((CB, D), jnp.float32),
        pltpu.VMEM((CB, D), jnp.float32),
        pltpu.VMEM((CB, D), jnp.float32),
        pltpu.VMEM((CB * K, D), jnp.float32),
        pltpu.VMEM((CB * K, D), jnp.float32),
        pltpu.VMEM((BPW,), jnp.float32),
        pltpu.VMEM((BPW * K,), jnp.float32),
        pltpu.SemaphoreType.DMA,
        pltpu.SemaphoreType.DMA,
    ],
    compiler_params=pltpu.CompilerParams(needs_layout_passes=False,
                                         use_tc_tiling_on_sc=False),
)
def _sc_scores(cw_hbm, pc_hbm, neg_hbm, wtab_hbm, ctab_hbm,
               pos_out, neg_out,
               cwi_v, pci_v, ngi_v, cwg0, cwg1, pcg0, pcg1, ngg0, ngg1,
               wbuf0, wbuf1, cbuf0, cbuf1, nbuf0, nbuf1,
               psc_v, nsc_v, sem0, sem1):
    wid = lax.axis_index("s") * NC + lax.axis_index("c")
    base = wid * BPW

    stage = [
        pltpu.async_copy(cw_hbm.at[pl.ds(base, BPW)], cwi_v, sem0),
        pltpu.async_copy(pc_hbm.at[pl.ds(base, BPW)], pci_v, sem0),
    ]
    stage += [
        pltpu.async_copy(neg_hbm.at[pl.ds(k * B + base, BPW)],
                         ngi_v.at[pl.ds(k * BPW, BPW)], sem0)
        for k in range(K)
    ]
    for c in stage:
        c.wait()

    lanes = lax.iota(jnp.int32, CB)
    bufs = (
        (cwg0, pcg0, ngg0, wbuf0, cbuf0, nbuf0, sem0),
        (cwg1, pcg1, ngg1, wbuf1, cbuf1, nbuf1, sem1),
    )

    def grow(i):
        return ((i & (SLINES - 1)) << 3) | lax.shift_right_logical(i, LOGS)

    def fire(ch, slot):
        cwg, pcg, ngg, wb, cb_, nb, sem = bufs[slot]
        cb = ch * CB
        cwg[...] = grow(cwi_v[pl.ds(cb, CB)])
        pcg[...] = grow(pci_v[pl.ds(cb, CB)])
        for k in range(K):
            ngg[pl.ds(k * CB, CB)] = \
                grow(ngi_v[pl.ds(k * BPW + cb, CB)])
        return (pltpu.async_copy(wtab_hbm.at[cwg], wb, sem),
                pltpu.async_copy(ctab_hbm.at[pcg], cb_, sem),
                pltpu.async_copy(ctab_hbm.at[ngg], nb, sem))

    def compute(ch, slot, copies):
        _, _, _, wb, cb_, nb, sem = bufs[slot]
        for c in copies:
            c.wait()
        cb = ch * CB
        acc = jnp.zeros((CB,), jnp.float32)
        wj = []
        for j in range(D):
            jv = jnp.full((CB,), j, jnp.int32)
            wv = plsc.load_gather(wb, [lanes, jv])
            cv = plsc.load_gather(cb_, [lanes, jv])
            wj.append(wv)
            acc = acc + wv * cv
        psc_v[pl.ds(cb, CB)] = acc

        for k in range(K):
            rows_nk = k * CB + lanes
            accn = jnp.zeros((CB,), jnp.float32)
            for j in range(D):
                nv = plsc.load_gather(nb, [rows_nk, jnp.full((CB,), j, jnp.int32)])
                accn = accn + nv * wj[j]
            nsc_v[pl.ds(cb * K + k * CB, CB)] = accn

    def pair(g, carry):
        ch = g * 2
        c_a = fire(ch, 0)
        c_b = fire(ch + 1, 1)
        compute(ch, 0, c_a)
        compute(ch + 1, 1, c_b)
        return carry

    lax.fori_loop(0, NCHUNK // 2, pair, 0)

    pltpu.sync_copy(psc_v, pos_out.at[pl.ds(base, BPW)])
    pltpu.sync_copy(nsc_v, neg_out.at[pl.ds(base * K, BPW * K)])


TW = 4096
TGRID = SLINES // TW


def _tr_body(*refs):
    wrefs = refs[:RPL]
    crefs = refs[RPL:2 * RPL]
    wo_ref, co_ref = refs[2 * RPL:]
    ident = jnp.eye(128, dtype=jnp.float32)
    for srcs, dst in ((wrefs, wo_ref), (crefs, co_ref)):
        xs = jnp.concatenate([r[...] for r in srcs], axis=0)
        dst[...] = lax.dot_general(xs, ident, (((0,), (0,)), ((), ())))


def _transpose_tables(wt_t, ct_t):
    last = 1000000 // TW
    in_spec = [
        pl.BlockSpec((D, TW), lambda g, a=a: (0, jnp.minimum(a * TGRID + g, last)))
        for a in range(RPL)
    ]
    return pl.pallas_call(
        _tr_body,
        grid=(TGRID,),
        in_specs=in_spec + in_spec,
        out_specs=[
            pl.BlockSpec((TW, 128), lambda g: (g, 0)),
            pl.BlockSpec((TW, 128), lambda g: (g, 0)),
        ],
        out_shape=[
            jax.ShapeDtypeStruct((SLINES, 128), jnp.float32),
            jax.ShapeDtypeStruct((SLINES, 128), jnp.float32),
        ],
    )(*([wt_t] * RPL + [ct_t] * RPL))


def _log_sigmoid(x):
    return jnp.minimum(x, 0.0) - jnp.log1p(jnp.exp(-jnp.abs(x)))


def _tc_body(p_ref, n_ref, out_ref):
    s = jnp.sum(_log_sigmoid(p_ref[...])) + jnp.sum(_log_sigmoid(-n_ref[...]))
    out_ref[...] = (s * (-1.0 / B)).reshape(1, 1)


def _tc_loss(pos_score, neg_score):
    return pl.pallas_call(
        _tc_body,
        out_shape=jax.ShapeDtypeStruct((1, 1), jnp.float32),
    )(pos_score.reshape(B // 128, 128), neg_score.reshape(B * K // 128, 128))


def kernel(centrals_words, pos_context, neg_context, word_emb, con_emb):
    neg_km = neg_context.T.reshape(B * K)
    wt, ct = _transpose_tables(word_emb.T, con_emb.T)
    wt16 = wt.reshape(SLINES * RPL, D)
    ct16 = ct.reshape(SLINES * RPL, D)
    pos_score, neg_score = _sc_scores(centrals_words, pos_context, neg_km,
                                      wt16, ct16)
    loss = _tc_loss(pos_score, neg_score)
    return loss[0, 0]


# speedup vs baseline: 6.0319x; 1.0053x over previous
import functools

import jax
import jax.numpy as jnp
from jax import lax
from jax.experimental import pallas as pl
from jax.experimental.pallas import tpu as pltpu
from jax.experimental.pallas import tpu_sc as plsc

B = 16384
K = 20
D = 16
RPL = 128 // D
SLINES = 131072
LOGS = 17
NC = 2
NS = 16
NW = NC * NS
BPW = B // NW
CB = 32
NCHUNK = BPW // CB

_mesh = plsc.VectorSubcoreMesh(core_axis_name="c", subcore_axis_name="s")


@functools.partial(
    pl.kernel,
    mesh=_mesh,
    out_type=(
        jax.ShapeDtypeStruct((B,), jnp.float32),
        jax.ShapeDtypeStruct((B * K,), jnp.float32),
    ),
    scratch_types=[
        pltpu.VMEM((BPW,), jnp.int32),
        pltpu.VMEM((BPW,), jnp.int32),
        pltpu.VMEM((BPW * K,), jnp.int32),
        pltpu.VMEM((CB,), jnp.int32),
        pltpu.VMEM((CB,), jnp.int32),
        pltpu.VMEM((CB,), jnp.int32),
        pltpu.VMEM((CB,), jnp.int32),
        pltpu.VMEM((CB * K,), jnp.int32),
        pltpu.VMEM((CB * K,), jnp.int32),
        pltpu.VMEM((CB, D), jnp.float32),
        pltpu.VMEM((CB, D), jnp.float32),
        pltpu.VMEM((CB, D), jnp.float32),
        pltpu.VMEM((CB, D), jnp.float32),
        pltpu.VMEM((CB * K, D), jnp.float32),
        pltpu.VMEM((CB * K, D), jnp.float32),
        pltpu.VMEM((BPW,), jnp.float32),
        pltpu.VMEM((BPW * K,), jnp.float32),
        pltpu.SemaphoreType.DMA,
        pltpu.SemaphoreType.DMA,
    ],
    compiler_params=pltpu.CompilerParams(needs_layout_passes=False,
                                         use_tc_tiling_on_sc=False),
)
def _sc_scores(cw_hbm, pc_hbm, neg_hbm, wtab_hbm, ctab_hbm,
               pos_out, neg_out,
               cwi_v, pci_v, ngi_v, cwg0, cwg1, pcg0, pcg1, ngg0, ngg1,
               wbuf0, wbuf1, cbuf0, cbuf1, nbuf0, nbuf1,
               psc_v, nsc_v, sem0, sem1):
    wid = lax.axis_index("s") * NC + lax.axis_index("c")
    base = wid * BPW

    stage = [
        pltpu.async_copy(cw_hbm.at[pl.ds(base, BPW)], cwi_v, sem0),
        pltpu.async_copy(pc_hbm.at[pl.ds(base, BPW)], pci_v, sem0),
    ]
    stage += [
        pltpu.async_copy(neg_hbm.at[pl.ds(k * B + base, BPW)],
                         ngi_v.at[pl.ds(k * BPW, BPW)], sem0)
        for k in range(K)
    ]
    for c in stage:
        c.wait()

    lanes = lax.iota(jnp.int32, 16)
    bufs = (
        (cwg0, pcg0, ngg0, wbuf0, cbuf0, nbuf0, sem0),
        (cwg1, pcg1, ngg1, wbuf1, cbuf1, nbuf1, sem1),
    )

    def grow(i):
        return ((i & (SLINES - 1)) << 3) | lax.shift_right_logical(i, LOGS)

    def fire(ch, slot):
        cwg, pcg, ngg, wb, cb_, nb, sem = bufs[slot]
        cb = ch * CB
        for g2 in range(CB // 16):
            o = g2 * 16
            cwg[pl.ds(o, 16)] = grow(cwi_v[pl.ds(cb + o, 16)])
            pcg[pl.ds(o, 16)] = grow(pci_v[pl.ds(cb + o, 16)])
            for k in range(K):
                ngg[pl.ds(k * CB + o, 16)] = \
                    grow(ngi_v[pl.ds(k * BPW + cb + o, 16)])
        return (pltpu.async_copy(wtab_hbm.at[cwg], wb, sem),
                pltpu.async_copy(ctab_hbm.at[pcg], cb_, sem),
                pltpu.async_copy(ctab_hbm.at[ngg], nb, sem))

    def compute(ch, slot, copies):
        _, _, _, wb, cb_, nb, sem = bufs[slot]
        for c in copies:
            c.wait()
        cb = ch * CB
        for g2 in range(CB // 16):
            o = g2 * 16
            rows_w = o + lanes
            acc = jnp.zeros((16,), jnp.float32)
            wj = []
            for j in range(D):
                jv = jnp.full((16,), j, jnp.int32)
                wv = plsc.load_gather(wb, [rows_w, jv])
                cv = plsc.load_gather(cb_, [rows_w, jv])
                wj.append(wv)
                acc = acc + wv * cv
            psc_v[pl.ds(cb + o, 16)] = acc

            for k in range(K):
                rows_nk = k * CB + o + lanes
                accn = jnp.zeros((16,), jnp.float32)
                for j in range(D):
                    nv = plsc.load_gather(
                        nb, [rows_nk, jnp.full((16,), j, jnp.int32)])
                    accn = accn + nv * wj[j]
                nsc_v[pl.ds(cb * K + k * CB + o, 16)] = accn

    def pair(g, carry):
        ch = g * 2
        c_a = fire(ch, 0)
        c_b = fire(ch + 1, 1)
        compute(ch, 0, c_a)
        compute(ch + 1, 1, c_b)
        return carry

    lax.fori_loop(0, NCHUNK // 2, pair, 0)

    pltpu.sync_copy(psc_v, pos_out.at[pl.ds(base, BPW)])
    pltpu.sync_copy(nsc_v, neg_out.at[pl.ds(base * K, BPW * K)])


TW = 4096
TGRID = SLINES // TW


def _tr_body(*refs):
    wrefs = refs[:RPL]
    crefs = refs[RPL:2 * RPL]
    wo_ref, co_ref = refs[2 * RPL:]
    ident = jnp.eye(128, dtype=jnp.float32)
    for srcs, dst in ((wrefs, wo_ref), (crefs, co_ref)):
        xs = jnp.concatenate([r[...] for r in srcs], axis=0)
        dst[...] = lax.dot_general(xs, ident, (((0,), (0,)), ((), ())))


def _transpose_tables(wt_t, ct_t):
    last = 1000000 // TW
    in_spec = [
        pl.BlockSpec((D, TW), lambda g, a=a: (0, jnp.minimum(a * TGRID + g, last)))
        for a in range(RPL)
    ]
    return pl.pallas_call(
        _tr_body,
        grid=(TGRID,),
        in_specs=in_spec + in_spec,
        out_specs=[
            pl.BlockSpec((TW, 128), lambda g: (g, 0)),
            pl.BlockSpec((TW, 128), lambda g: (g, 0)),
        ],
        out_shape=[
            jax.ShapeDtypeStruct((SLINES, 128), jnp.float32),
            jax.ShapeDtypeStruct((SLINES, 128), jnp.float32),
        ],
    )(*([wt_t] * RPL + [ct_t] * RPL))


def _log_sigmoid(x):
    return jnp.minimum(x, 0.0) - jnp.log1p(jnp.exp(-jnp.abs(x)))


def _tc_body(p_ref, n_ref, out_ref):
    s = jnp.sum(_log_sigmoid(p_ref[...])) + jnp.sum(_log_sigmoid(-n_ref[...]))
    out_ref[...] = (s * (-1.0 / B)).reshape(1, 1)


def _tc_loss(pos_score, neg_score):
    return pl.pallas_call(
        _tc_body,
        out_shape=jax.ShapeDtypeStruct((1, 1), jnp.float32),
    )(pos_score.reshape(B // 128, 128), neg_score.reshape(B * K // 128, 128))


def kernel(centrals_words, pos_context, neg_context, word_emb, con_emb):
    neg_km = neg_context.T.reshape(B * K)
    wt, ct = _transpose_tables(word_emb.T, con_emb.T)
    wt16 = wt.reshape(SLINES * RPL, D)
    ct16 = ct.reshape(SLINES * RPL, D)
    pos_score, neg_score = _sc_scores(centrals_words, pos_context, neg_km,
                                      wt16, ct16)
    loss = _tc_loss(pos_score, neg_score)
    return loss[0, 0]
